# baseline (speedup 1.0000x reference)
import jax
import jax.numpy as jnp
from jax import lax
from jax.experimental import pallas as pl
from jax.experimental.pallas import tpu as pltpu
from jax.experimental.pallas import tpu_sc as plsc

N = 10000
E = 320000
D = 128

NC = 2
NS = 16
NW = NC * NS

K = 128
CH_TOT = E // K
CB = CH_TOT // NW
XTRA = CH_TOT - CB * NW
PAIRS = (CB - 2) // 2
N_PAD = 10240
R_T = N_PAD // NS

_mesh = plsc.VectorSubcoreMesh(core_axis_name="c", subcore_axis_name="s")


def _fill(ref, n, val, dtype):
    v = jnp.full((16,), val, dtype=dtype)

    def body(i, _):
        ref[pl.ds(i * 16, 16)] = v
        return 0

    lax.fori_loop(0, n // 16, body, 0)


def _fill2d(ref, rows, cols, val, dtype):
    v = jnp.full((16,), val, dtype=dtype)

    def body(i, _):
        r = i // (cols // 16)
        c = (i % (cols // 16)) * 16
        ref[r, pl.ds(c, 16)] = v
        return 0

    lax.fori_loop(0, rows * (cols // 16), body, 0)


def _tile_range(wid):
    start = CB * wid + jnp.minimum(wid, XTRA)
    n = jnp.where(wid < XTRA, CB + 1, CB)
    return start, n


def _deg_body(dst_hbm, out_hbm, didx_all, ones_v, buf_v, sem, deg_sh):
    cid = lax.axis_index("c")
    sid = lax.axis_index("s")
    wid = cid * NS + sid
    start, n = _tile_range(wid)

    _fill(ones_v, K, 1.0, jnp.float32)
    _fill(buf_v, R_T, 0.0, jnp.float32)
    pltpu.sync_copy(dst_hbm.at[pl.ds(start, CB)], didx_all.at[pl.ds(0, CB)])
    last = jnp.minimum(start + CB, CH_TOT - 1)
    pltpu.sync_copy(dst_hbm.at[pl.ds(last, 1)], didx_all.at[pl.ds(CB, 1)])
    pltpu.sync_copy(buf_v, deg_sh.at[pl.ds(sid * R_T, R_T)])
    plsc.subcore_barrier()

    def fire(i, _):
        pltpu.async_copy(ones_v, deg_sh.at[didx_all.at[i, 0]], sem, add=True)
        return 0

    def drain(i, _):
        pltpu.make_async_copy(ones_v, deg_sh.at[didx_all.at[i, 0]], sem).wait()
        return 0

    lax.fori_loop(0, n, fire, 0)
    lax.fori_loop(0, n, drain, 0)

    plsc.subcore_barrier()
    pltpu.sync_copy(deg_sh.at[pl.ds(sid * R_T, R_T)],
                    out_hbm.at[cid, pl.ds(sid * R_T, R_T)])


_deg_call = pl.kernel(
    _deg_body,
    out_type=jax.ShapeDtypeStruct((NC, N_PAD), jnp.float32),
    mesh=_mesh,
    scratch_types=[
        pltpu.VMEM((CB + 1, 1, K), jnp.int32),
        pltpu.VMEM((K,), jnp.float32),
        pltpu.VMEM((R_T,), jnp.float32),
        pltpu.SemaphoreType.DMA,
        pltpu.VMEM_SHARED((N_PAD,), jnp.float32),
    ],
)


def _agg_body(src_hbm, dst_hbm, y_hbm, out_hbm, sidx, didx,
              rows_a, rows_b, sem_a, sem_b, sem_i, acc_sh):
    cid = lax.axis_index("c")
    sid = lax.axis_index("s")
    wid = cid * NS + sid
    start, n = _tile_range(wid)
    n79 = n > CB

    _fill2d(rows_a, K, D, 0.0, jnp.float32)
    for j in range(R_T // K):
        pltpu.sync_copy(rows_a, acc_sh.at[pl.ds(sid * R_T + j * K, K)])
    plsc.subcore_barrier()

    def idx_row(c):
        return jnp.minimum(start + c, CH_TOT - 1)

    def stage_sync(c):
        sl = lax.rem(c, 4)
        pltpu.sync_copy(src_hbm.at[idx_row(c)], sidx.at[pl.ds(sl, 1)])
        pltpu.sync_copy(dst_hbm.at[idx_row(c)], didx.at[pl.ds(sl, 1)])

    def stage_async(c):
        sl = lax.rem(c, 4)
        pltpu.async_copy(src_hbm.at[idx_row(c)], sidx.at[pl.ds(sl, 1)],
                         sem_i)
        pltpu.async_copy(dst_hbm.at[idx_row(c)], didx.at[pl.ds(sl, 1)],
                         sem_i)

    def stage_drain(c):
        sl = lax.rem(c, 4)
        pltpu.make_async_copy(src_hbm.at[idx_row(c)],
                              sidx.at[pl.ds(sl, 1)], sem_i).wait()
        pltpu.make_async_copy(dst_hbm.at[idx_row(c)],
                              didx.at[pl.ds(sl, 1)], sem_i).wait()

    def g_start(c, buf, sem):
        pltpu.async_copy(y_hbm.at[sidx.at[lax.rem(c, 4)]], buf, sem)

    def g_wait(c, buf, sem):
        pltpu.make_async_copy(y_hbm.at[sidx.at[lax.rem(c, 4)]], buf,
                              sem).wait()

    def scat(c, buf):
        pltpu.sync_copy(buf, acc_sh.at[didx.at[lax.rem(c, 4)]], add=True)

    for c in range(4):
        stage_sync(jnp.int32(c))
    g_start(jnp.int32(0), rows_a, sem_a)
    g_start(jnp.int32(1), rows_b, sem_b)

    def pair(i, _):
        c0 = 2 * i
        g_wait(c0, rows_a, sem_a)
        scat(c0, rows_a)
        g_start(c0 + 2, rows_a, sem_a)
        g_wait(c0 + 1, rows_b, sem_b)
        scat(c0 + 1, rows_b)
        g_start(c0 + 3, rows_b, sem_b)
        stage_async(c0 + 4)
        stage_async(c0 + 5)
        stage_drain(c0 + 4)
        stage_drain(c0 + 5)
        return 0

    lax.fori_loop(0, PAIRS, pair, 0)

    cl = jnp.int32(2 * PAIRS)
    g_wait(cl, rows_a, sem_a)
    scat(cl, rows_a)

    @pl.when(n79)
    def _():
        g_start(cl + 2, rows_a, sem_a)

    g_wait(cl + 1, rows_b, sem_b)
    scat(cl + 1, rows_b)

    @pl.when(n79)
    def _():
        g_wait(cl + 2, rows_a, sem_a)
        scat(cl + 2, rows_a)

    plsc.subcore_barrier()
    pltpu.sync_copy(acc_sh.at[pl.ds(sid * R_T, R_T)],
                    out_hbm.at[cid, pl.ds(sid * R_T, R_T)])


_agg_call = pl.kernel(
    _agg_body,
    out_type=jax.ShapeDtypeStruct((NC, N_PAD, D), jnp.float32),
    mesh=_mesh,
    scratch_types=[
        pltpu.VMEM((4, K), jnp.int32),
        pltpu.VMEM((4, K), jnp.int32),
        pltpu.VMEM((K, D), jnp.float32),
        pltpu.VMEM((K, D), jnp.float32),
        pltpu.SemaphoreType.DMA,
        pltpu.SemaphoreType.DMA,
        pltpu.SemaphoreType.DMA,
        pltpu.VMEM_SHARED((N_PAD, D), jnp.float32),
    ],
)


def _xw_body(x_ref, w1_ref, xw_ref):
    xw_ref[...] = lax.dot_general(
        x_ref[...], w1_ref[...], (((1,), (1,)), ((), ())),
        preferred_element_type=jnp.float32)


_xw_call = pl.pallas_call(
    _xw_body,
    out_shape=jax.ShapeDtypeStruct((N, D), jnp.float32),
)


def _scale_body(xw_ref, degp_ref, y_ref, dinv_ref):
    deg = degp_ref[0:1, :] + degp_ref[1:2, :] + 1.0
    dinv_row = lax.rsqrt(deg)
    dinv_col = dinv_row.reshape(N_PAD, 1)[:N, :]
    y_ref[...] = xw_ref[...] * dinv_col
    dinv_ref[...] = dinv_row


_scale_call = pl.pallas_call(
    _scale_body,
    out_shape=(
        jax.ShapeDtypeStruct((N, D), jnp.float32),
        jax.ShapeDtypeStruct((1, N_PAD), jnp.float32),
    ),
)


def _final_body(accp_ref, y_ref, dinv_ref, b1_ref, wl_ref, bl_ref,
                out_ref):
    dinv_col = dinv_ref[...].reshape(N_PAD, 1)[:N, :]
    agg = (accp_ref[0, pl.ds(0, N), :] + accp_ref[1, pl.ds(0, N), :]
           + y_ref[...])
    h = jnp.maximum(agg * dinv_col + b1_ref[...], 0.0)
    out_ref[...] = lax.dot_general(
        h, wl_ref[...], (((1,), (1,)), ((), ())),
        preferred_element_type=jnp.float32) + bl_ref[...]


_final_call = pl.pallas_call(
    _final_body,
    out_shape=jax.ShapeDtypeStruct((N, D), jnp.float32),
)


def kernel(x, edge_index, W1, b1, Wl, bl):
    e32 = edge_index.astype(jnp.int32)
    src3d = e32[0].reshape(CH_TOT, 1, K)
    dst3d = e32[1].reshape(CH_TOT, 1, K)

    degp = _deg_call(dst3d)
    xw = _xw_call(x, W1)
    y, dinv = _scale_call(xw, degp)
    accp = _agg_call(src3d, dst3d, y)
    out = _final_call(accp, y, dinv, b1[None, :], Wl, bl[None, :])
    return out


# speedup vs baseline: 1.0067x; 1.0067x over previous
import jax
import jax.numpy as jnp
from jax import lax
from jax.experimental import pallas as pl
from jax.experimental.pallas import tpu as pltpu
from jax.experimental.pallas import tpu_sc as plsc

N = 10000
E = 320000
D = 128

NC = 2
NS = 16
NW = NC * NS

K = 128
CH_TOT = E // K
CB = CH_TOT // NW
XTRA = CH_TOT - CB * NW
PAIRS = (CB - 2) // 2
N_PAD = 10240
R_T = N_PAD // NS

_mesh = plsc.VectorSubcoreMesh(core_axis_name="c", subcore_axis_name="s")


def _fill(ref, n, val, dtype):
    v = jnp.full((16,), val, dtype=dtype)

    def body(i, _):
        ref[pl.ds(i * 16, 16)] = v
        return 0

    lax.fori_loop(0, n // 16, body, 0)


def _fill2d(ref, rows, cols, val, dtype):
    v = jnp.full((16,), val, dtype=dtype)

    def body(i, _):
        r = i // (cols // 16)
        c = (i % (cols // 16)) * 16
        ref[r, pl.ds(c, 16)] = v
        return 0

    lax.fori_loop(0, rows * (cols // 16), body, 0)


def _tile_range(wid):
    start = CB * wid + jnp.minimum(wid, XTRA)
    n = jnp.where(wid < XTRA, CB + 1, CB)
    return start, n


def _deg_body(dst_hbm, out_hbm, didx_all, ones_v, buf_v, sem, deg_sh):
    cid = lax.axis_index("c")
    sid = lax.axis_index("s")
    wid = cid * NS + sid
    start, n = _tile_range(wid)

    _fill(ones_v, K, 1.0, jnp.float32)
    _fill(buf_v, R_T, 0.0, jnp.float32)
    pltpu.sync_copy(dst_hbm.at[pl.ds(start, CB)], didx_all.at[pl.ds(0, CB)])
    last = jnp.minimum(start + CB, CH_TOT - 1)
    pltpu.sync_copy(dst_hbm.at[pl.ds(last, 1)], didx_all.at[pl.ds(CB, 1)])
    pltpu.sync_copy(buf_v, deg_sh.at[pl.ds(sid * R_T, R_T)])
    plsc.subcore_barrier()

    def fire(i, _):
        pltpu.async_copy(ones_v, deg_sh.at[didx_all.at[i, 0]], sem, add=True)
        return 0

    def drain(i, _):
        pltpu.make_async_copy(ones_v, deg_sh.at[didx_all.at[i, 0]], sem).wait()
        return 0

    lax.fori_loop(0, n, fire, 0)
    lax.fori_loop(0, n, drain, 0)

    plsc.subcore_barrier()
    pltpu.sync_copy(deg_sh.at[pl.ds(sid * R_T, R_T)],
                    out_hbm.at[cid, pl.ds(sid * R_T, R_T)])


_deg_call = pl.kernel(
    _deg_body,
    out_type=jax.ShapeDtypeStruct((NC, N_PAD), jnp.float32),
    mesh=_mesh,
    scratch_types=[
        pltpu.VMEM((CB + 1, 1, K), jnp.int32),
        pltpu.VMEM((K,), jnp.float32),
        pltpu.VMEM((R_T,), jnp.float32),
        pltpu.SemaphoreType.DMA,
        pltpu.VMEM_SHARED((N_PAD,), jnp.float32),
    ],
)


def _agg_body(src_hbm, dst_hbm, y_hbm, out_hbm, sidx, didx,
              rows_a, rows_b, sem_a, sem_b, sem_i, acc_sh):
    cid = lax.axis_index("c")
    sid = lax.axis_index("s")
    wid = cid * NS + sid
    start, n = _tile_range(wid)
    n79 = n > CB

    _fill2d(rows_a, K, D, 0.0, jnp.float32)
    for j in range(R_T // K):
        pltpu.async_copy(rows_a, acc_sh.at[pl.ds(sid * R_T + j * K, K)],
                         sem_i)
    for j in range(R_T // K):
        pltpu.make_async_copy(rows_a,
                              acc_sh.at[pl.ds(sid * R_T + j * K, K)],
                              sem_i).wait()
    plsc.subcore_barrier()

    def idx_row(c):
        return jnp.minimum(start + c, CH_TOT - 1)

    def stage_sync(c):
        sl = lax.rem(c, 4)
        pltpu.sync_copy(src_hbm.at[idx_row(c)], sidx.at[pl.ds(sl, 1)])
        pltpu.sync_copy(dst_hbm.at[idx_row(c)], didx.at[pl.ds(sl, 1)])

    def stage_async(c):
        sl = lax.rem(c, 4)
        pltpu.async_copy(src_hbm.at[idx_row(c)], sidx.at[pl.ds(sl, 1)],
                         sem_i)
        pltpu.async_copy(dst_hbm.at[idx_row(c)], didx.at[pl.ds(sl, 1)],
                         sem_i)

    def stage_drain(c):
        sl = lax.rem(c, 4)
        pltpu.make_async_copy(src_hbm.at[idx_row(c)],
                              sidx.at[pl.ds(sl, 1)], sem_i).wait()
        pltpu.make_async_copy(dst_hbm.at[idx_row(c)],
                              didx.at[pl.ds(sl, 1)], sem_i).wait()

    def g_start(c, buf, sem):
        pltpu.async_copy(y_hbm.at[sidx.at[lax.rem(c, 4)]], buf, sem)

    def g_wait(c, buf, sem):
        pltpu.make_async_copy(y_hbm.at[sidx.at[lax.rem(c, 4)]], buf,
                              sem).wait()

    def scat(c, buf):
        pltpu.sync_copy(buf, acc_sh.at[didx.at[lax.rem(c, 4)]], add=True)

    for c in range(4):
        stage_sync(jnp.int32(c))
    g_start(jnp.int32(0), rows_a, sem_a)
    g_start(jnp.int32(1), rows_b, sem_b)

    def pair(i, _):
        c0 = 2 * i
        g_wait(c0, rows_a, sem_a)
        scat(c0, rows_a)
        g_start(c0 + 2, rows_a, sem_a)
        g_wait(c0 + 1, rows_b, sem_b)
        scat(c0 + 1, rows_b)
        g_start(c0 + 3, rows_b, sem_b)
        stage_async(c0 + 4)
        stage_async(c0 + 5)
        stage_drain(c0 + 4)
        stage_drain(c0 + 5)
        return 0

    lax.fori_loop(0, PAIRS, pair, 0)

    cl = jnp.int32(2 * PAIRS)
    g_wait(cl, rows_a, sem_a)
    scat(cl, rows_a)

    @pl.when(n79)
    def _():
        g_start(cl + 2, rows_a, sem_a)

    g_wait(cl + 1, rows_b, sem_b)
    scat(cl + 1, rows_b)

    @pl.when(n79)
    def _():
        g_wait(cl + 2, rows_a, sem_a)
        scat(cl + 2, rows_a)

    plsc.subcore_barrier()
    pltpu.sync_copy(acc_sh.at[pl.ds(sid * R_T, R_T)],
                    out_hbm.at[cid, pl.ds(sid * R_T, R_T)])


_agg_call = pl.kernel(
    _agg_body,
    out_type=jax.ShapeDtypeStruct((NC, N_PAD, D), jnp.float32),
    mesh=_mesh,
    scratch_types=[
        pltpu.VMEM((4, K), jnp.int32),
        pltpu.VMEM((4, K), jnp.int32),
        pltpu.VMEM((K, D), jnp.float32),
        pltpu.VMEM((K, D), jnp.float32),
        pltpu.SemaphoreType.DMA,
        pltpu.SemaphoreType.DMA,
        pltpu.SemaphoreType.DMA,
        pltpu.VMEM_SHARED((N_PAD, D), jnp.float32),
    ],
)


def _prep_body(x_ref, w1_ref, degp_ref, y_ref, dinv_ref):
    deg = degp_ref[0:1, :] + degp_ref[1:2, :] + 1.0
    dinv_row = lax.rsqrt(deg)
    dinv_col = dinv_row.reshape(N_PAD, 1)[:N, :]
    xw = lax.dot_general(
        x_ref[...], w1_ref[...], (((1,), (1,)), ((), ())),
        preferred_element_type=jnp.float32)
    y_ref[...] = xw * dinv_col
    dinv_ref[...] = dinv_row


_prep_call = pl.pallas_call(
    _prep_body,
    out_shape=(
        jax.ShapeDtypeStruct((N, D), jnp.float32),
        jax.ShapeDtypeStruct((1, N_PAD), jnp.float32),
    ),
)


def _final_body(accp_ref, y_ref, dinv_ref, b1_ref, wl_ref, bl_ref,
                out_ref):
    dinv_col = dinv_ref[...].reshape(N_PAD, 1)[:N, :]
    agg = (accp_ref[0, pl.ds(0, N), :] + accp_ref[1, pl.ds(0, N), :]
           + y_ref[...])
    h = jnp.maximum(agg * dinv_col + b1_ref[...], 0.0)
    out_ref[...] = lax.dot_general(
        h, wl_ref[...], (((1,), (1,)), ((), ())),
        preferred_element_type=jnp.float32) + bl_ref[...]


_final_call = pl.pallas_call(
    _final_body,
    out_shape=jax.ShapeDtypeStruct((N, D), jnp.float32),
)


def kernel(x, edge_index, W1, b1, Wl, bl):
    e32 = edge_index.astype(jnp.int32)
    src3d = e32[0].reshape(CH_TOT, 1, K)
    dst3d = e32[1].reshape(CH_TOT, 1, K)

    degp = _deg_call(dst3d)
    y, dinv = _prep_call(x, W1, degp)
    accp = _agg_call(src3d, dst3d, y)
    out = _final_call(accp, y, dinv, b1[None, :], Wl, bl[None, :])
    return out
